# baseline (speedup 1.0000x reference)
import functools

import jax
import jax.numpy as jnp
import numpy as np
from jax import lax
from jax.experimental import pallas as pl
from jax.experimental.pallas import tpu as pltpu
from jax.experimental.pallas import tpu_sc as plsc

OUT_HW = 7
SR = 2
S = OUT_HW * SR
SCALE = 0.25
NB, C, H, W = 2, 256, 100, 100
R = 1000
BINS = OUT_HW * OUT_HW
K = 16
J = R * BINS
NW = 32
RPAD = 1024
RPW = RPAD // NW
CB = OUT_HW
CPW = RPW * OUT_HW
NPAIR = CPW // 2
CG = C // 32


def _selection_mats():
    my = np.zeros((2 * S, BINS * K), np.float32)
    mx = np.zeros((2 * S, BINS * K), np.float32)
    for p in range(OUT_HW):
        for q in range(OUT_HW):
            for i in range(SR):
                for jj in range(SR):
                    for cy in range(2):
                        for cx in range(2):
                            col = (p * OUT_HW + q) * K + (i * SR + jj) * 4 + cy * 2 + cx
                            my[cy * S + (SR * p + i), col] = 1.0
                            mx[cx * S + (SR * q + jj), col] = 1.0
    return my, mx


_MY, _MX = _selection_mats()


def _coef_body(rois_ref, my_ref, mx_ref, w_ref, idx_ref):
    r = rois_ref[:]
    b = r[:, 0:1]
    x1 = r[:, 1:2] * SCALE
    y1 = r[:, 2:3] * SCALE
    x2 = r[:, 3:4] * SCALE
    y2 = r[:, 4:5] * SCALE
    bin_w = jnp.maximum(x2 - x1, 1.0) / OUT_HW
    bin_h = jnp.maximum(y2 - y1, 1.0) / OUT_HW
    s = lax.broadcasted_iota(jnp.int32, (1, S), 1).astype(jnp.float32)
    p_ = jnp.floor(s * 0.5)
    off = p_ + ((s - 2.0 * p_) + 0.5) * 0.5
    gx = x1 + off * bin_w
    gy = y1 + off * bin_h

    def axis(coord, size):
        v = ((coord >= -1.0) & (coord <= float(size))).astype(jnp.float32)
        c = jnp.clip(coord, 0.0, float(size - 1))
        lo = jnp.minimum(jnp.floor(c), float(size - 2))
        f = c - lo
        return (jnp.concatenate([(1.0 - f) * v, f * v], axis=1),
                jnp.concatenate([lo, lo + 1.0], axis=1))

    wyc, iyc = axis(gy, H)
    wxc, ixc = axis(gx, W)
    my = my_ref[:]
    mx = mx_ref[:]
    wy_sel = jnp.dot(wyc, my, preferred_element_type=jnp.float32)
    wx_sel = jnp.dot(wxc, mx, preferred_element_type=jnp.float32)
    iy_sel = jnp.dot(iyc, my, preferred_element_type=jnp.float32)
    ix_sel = jnp.dot(ixc, mx, preferred_element_type=jnp.float32)
    w_ref[:] = 0.25 * wy_sel * wx_sel
    idx_ref[:] = (b * float(H * W) + iy_sel * float(W) + ix_sel).astype(jnp.int32)


def _coefs(rois, my, mx):
    return pl.pallas_call(
        _coef_body,
        out_shape=[jax.ShapeDtypeStruct((R, BINS * K), jnp.float32),
                   jax.ShapeDtypeStruct((R, BINS * K), jnp.int32)],
    )(rois, my, mx)


def _sc_gather(table, idxs, ws):
    mesh = plsc.VectorSubcoreMesh(core_axis_name="c", subcore_axis_name="s")

    @functools.partial(
        pl.kernel,
        out_type=jax.ShapeDtypeStruct((RPAD * C * BINS,), jnp.float32),
        mesh=mesh,
        compiler_params=pltpu.CompilerParams(needs_layout_passes=False),
        scratch_types=[
            pltpu.VMEM((CPW, CB * K), jnp.int32),
            pltpu.VMEM((CPW * CB * K,), jnp.float32),
            pltpu.VMEM((CB * K, C // 2), jnp.int32),
            pltpu.VMEM((CB * K, C // 2), jnp.int32),
            pltpu.VMEM((2 * C * BINS,), jnp.float32),
            pltpu.SemaphoreType.DMA,
            pltpu.SemaphoreType.DMA,
            pltpu.SemaphoreType.DMA,
            pltpu.SemaphoreType.DMA,
        ],
    )
    def k(table_hbm, idx_hbm, w_hbm, out_hbm, idx_v, w_v,
          rows0, rows1, outb, sg0, sg1, sw0, sw1):
        wid = lax.axis_index("s") * 2 + lax.axis_index("c")
        pltpu.sync_copy(idx_hbm.at[pl.ds(wid * CPW, CPW)], idx_v)
        pltpu.sync_copy(w_hbm.at[pl.ds(wid * CPW * CB * K, CPW * CB * K)], w_v)
        pltpu.async_copy(table_hbm.at[idx_v.at[0]], rows0, sg0)

        lane = lax.broadcasted_iota(jnp.int32, (16,), 0)
        cvec = [lane * (2 * BINS) + cc * (32 * BINS) for cc in range(CG)]

        def compute(g, rows_v):
            pdyn = g % OUT_HW
            half = (g // OUT_HW) % 2

            def bin_body(bb, c2):
                base = bb * K
                wbase = g * (CB * K) + base
                acc = [jnp.zeros((16,), jnp.float32) for _ in range(2 * CG)]
                for kk in range(K):
                    wspl = plsc.load_gather(
                        w_v, [jnp.full((16,), wbase + kk, jnp.int32)])
                    for cc in range(CG):
                        ev, od = plsc.unpack(
                            plsc.bitcast(rows_v[base + kk, pl.ds(cc * 16, 16)],
                                         jnp.bfloat16),
                            format=plsc.PackFormat.INTERLEAVED)
                        acc[2 * cc] = acc[2 * cc] + wspl * ev
                        acc[2 * cc + 1] = acc[2 * cc + 1] + wspl * od
                bsplat = jnp.full(
                    (16,), half * (C * BINS) + pdyn * OUT_HW + bb, jnp.int32)
                for cc in range(CG):
                    pos = cvec[cc] + bsplat
                    plsc.store_scatter(outb, [pos], acc[2 * cc])
                    plsc.store_scatter(outb, [pos + BINS], acc[2 * cc + 1])
                return c2

            lax.fori_loop(0, CB, bin_body, 0)

        def edges(g):
            par = (g // OUT_HW) % 2

            def pre():
                @pl.when(jnp.logical_and(g >= 2 * OUT_HW, g % OUT_HW == 0))
                def _():
                    @pl.when(par == 0)
                    def _():
                        pltpu.make_async_copy(
                            outb.at[pl.ds(0, C * BINS)],
                            out_hbm.at[pl.ds(0, C * BINS)], sw0).wait()

                    @pl.when(par == 1)
                    def _():
                        pltpu.make_async_copy(
                            outb.at[pl.ds(0, C * BINS)],
                            out_hbm.at[pl.ds(0, C * BINS)], sw1).wait()

            def post():
                ro = wid * RPW + g // OUT_HW

                @pl.when(g % OUT_HW == OUT_HW - 1)
                def _():
                    @pl.when(par == 0)
                    def _():
                        pltpu.async_copy(
                            outb.at[pl.ds(0, C * BINS)],
                            out_hbm.at[pl.ds(ro * C * BINS, C * BINS)], sw0)

                    @pl.when(par == 1)
                    def _():
                        pltpu.async_copy(
                            outb.at[pl.ds(C * BINS, C * BINS)],
                            out_hbm.at[pl.ds(ro * C * BINS, C * BINS)], sw1)

            return pre, post

        def pair_body(t, carry):
            g0 = t * 2
            pltpu.async_copy(table_hbm.at[idx_v.at[g0 + 1]], rows1, sg1)
            pltpu.make_async_copy(table_hbm.at[idx_v.at[0]], rows0, sg0).wait()
            pre0, post0 = edges(g0)
            pre0()
            compute(g0, rows0)
            post0()

            @pl.when(t < NPAIR - 1)
            def _():
                pltpu.async_copy(table_hbm.at[idx_v.at[g0 + 2]], rows0, sg0)

            pltpu.make_async_copy(table_hbm.at[idx_v.at[0]], rows1, sg1).wait()
            pre1, post1 = edges(g0 + 1)
            pre1()
            compute(g0 + 1, rows1)
            post1()
            return carry

        lax.fori_loop(0, NPAIR, pair_body, 0)
        pltpu.make_async_copy(outb.at[pl.ds(0, C * BINS)],
                              out_hbm.at[pl.ds(0, C * BINS)], sw0).wait()
        pltpu.make_async_copy(outb.at[pl.ds(0, C * BINS)],
                              out_hbm.at[pl.ds(0, C * BINS)], sw1).wait()

    return k(table, idxs, ws)


def kernel(_input, rois):
    f = jnp.transpose(_input, (0, 2, 3, 1)).reshape(NB * H * W, C)
    f = lax.bitcast_convert_type(
        f.astype(jnp.bfloat16).reshape(NB * H * W, C // 2, 2), jnp.int32)
    w2, idx2 = _coefs(rois, jnp.asarray(_MY), jnp.asarray(_MX))

    def reorder(a):
        a = jnp.pad(a, ((0, RPAD - R), (0, 0)))
        return a.reshape(NW * CPW, CB * K)

    idx_2d = reorder(idx2)
    w_flat = reorder(w2).reshape(NW * CPW * CB * K)
    out_rows = _sc_gather(f, idx_2d, w_flat)
    return out_rows.reshape(RPAD, C, BINS)[:R].reshape(R, C, OUT_HW, OUT_HW)


# speedup vs baseline: 2.4213x; 2.4213x over previous
import functools

import jax
import jax.numpy as jnp
import numpy as np
from jax import lax
from jax.experimental import pallas as pl
from jax.experimental.pallas import tpu as pltpu
from jax.experimental.pallas import tpu_sc as plsc

OUT_HW = 7
SR = 2
S = OUT_HW * SR
SCALE = 0.25
NB, C, H, W = 2, 256, 100, 100
R = 1000
BINS = OUT_HW * OUT_HW
K = 16
J = R * BINS
NW = 32
JPAD = 49152
BINS_PER_W = JPAD // NW
CHUNK = 8
NCHUNK = BINS_PER_W // CHUNK
CG = C // 32
TR = 8


def _selection_mats():
    my = np.zeros((2 * S, BINS * K), np.float32)
    mx = np.zeros((2 * S, BINS * K), np.float32)
    for p in range(OUT_HW):
        for q in range(OUT_HW):
            for i in range(SR):
                for jj in range(SR):
                    for cy in range(2):
                        for cx in range(2):
                            col = (p * OUT_HW + q) * K + (i * SR + jj) * 4 + cy * 2 + cx
                            my[cy * S + (SR * p + i), col] = 1.0
                            mx[cx * S + (SR * q + jj), col] = 1.0
    return my, mx


_MY, _MX = _selection_mats()


def _coef_body(rois_ref, my_ref, mx_ref, w_ref, idx_ref):
    r = rois_ref[:]
    b = r[:, 0:1]
    x1 = r[:, 1:2] * SCALE
    y1 = r[:, 2:3] * SCALE
    x2 = r[:, 3:4] * SCALE
    y2 = r[:, 4:5] * SCALE
    bin_w = jnp.maximum(x2 - x1, 1.0) / OUT_HW
    bin_h = jnp.maximum(y2 - y1, 1.0) / OUT_HW
    s = lax.broadcasted_iota(jnp.int32, (1, S), 1).astype(jnp.float32)
    p_ = jnp.floor(s * 0.5)
    off = p_ + ((s - 2.0 * p_) + 0.5) * 0.5
    gx = x1 + off * bin_w
    gy = y1 + off * bin_h

    def axis(coord, size):
        v = ((coord >= -1.0) & (coord <= float(size))).astype(jnp.float32)
        c = jnp.clip(coord, 0.0, float(size - 1))
        lo = jnp.minimum(jnp.floor(c), float(size - 2))
        f = c - lo
        return (jnp.concatenate([(1.0 - f) * v, f * v], axis=1),
                jnp.concatenate([lo, lo + 1.0], axis=1))

    wyc, iyc = axis(gy, H)
    wxc, ixc = axis(gx, W)
    my = my_ref[:]
    mx = mx_ref[:]
    wy_sel = jnp.dot(wyc, my, preferred_element_type=jnp.float32)
    wx_sel = jnp.dot(wxc, mx, preferred_element_type=jnp.float32)
    iy_sel = jnp.dot(iyc, my, preferred_element_type=jnp.float32)
    ix_sel = jnp.dot(ixc, mx, preferred_element_type=jnp.float32)
    w_ref[:] = 0.25 * wy_sel * wx_sel
    idx_ref[:] = (b * float(H * W) + iy_sel * float(W) + ix_sel).astype(jnp.int32)


def _coefs(rois, my, mx):
    return pl.pallas_call(
        _coef_body,
        out_shape=[jax.ShapeDtypeStruct((R, BINS * K), jnp.float32),
                   jax.ShapeDtypeStruct((R, BINS * K), jnp.int32)],
    )(rois, my, mx)


def _sc_gather(table, idxs, ws):
    mesh = plsc.VectorSubcoreMesh(core_axis_name="c", subcore_axis_name="s")

    @functools.partial(
        pl.kernel,
        out_type=jax.ShapeDtypeStruct((JPAD * C,), jnp.float32),
        mesh=mesh,
        compiler_params=pltpu.CompilerParams(needs_layout_passes=False),
        scratch_types=[
            pltpu.VMEM((NCHUNK, CHUNK * K), jnp.int32),
            pltpu.VMEM((BINS_PER_W * K,), jnp.float32),
            pltpu.VMEM((CHUNK * K, C // 2), jnp.int32),
            pltpu.VMEM((CHUNK * K, C // 2), jnp.int32),
            pltpu.VMEM((CHUNK * C,), jnp.float32),
            pltpu.VMEM((CHUNK * C,), jnp.float32),
            pltpu.SemaphoreType.DMA,
            pltpu.SemaphoreType.DMA,
            pltpu.SemaphoreType.DMA,
            pltpu.SemaphoreType.DMA,
        ],
    )
    def k(table_hbm, idx_hbm, w_hbm, out_hbm, idx_v, w_v,
          rows0, rows1, out0, out1, sg0, sg1, sw0, sw1):
        wid = lax.axis_index("s") * 2 + lax.axis_index("c")
        tile0 = wid * BINS_PER_W
        pltpu.sync_copy(idx_hbm.at[pl.ds(wid * NCHUNK, NCHUNK)], idx_v)
        pltpu.sync_copy(w_hbm.at[pl.ds(tile0 * K, BINS_PER_W * K)], w_v)
        pltpu.async_copy(table_hbm.at[idx_v.at[0]], rows0, sg0)

        lane = lax.broadcasted_iota(jnp.int32, (16,), 0)

        def compute(g, rows_v, out_v, sw):
            bin0 = tile0 + g * CHUNK
            wbase = g * (CHUNK * K)

            def bin_body(bb, c2):
                base = bb * K
                acc = [jnp.zeros((16,), jnp.float32) for _ in range(2 * CG)]
                for kk in range(K):
                    wspl = plsc.load_gather(
                        w_v, [jnp.full((16,), wbase + base + kk, jnp.int32)])
                    for cc in range(CG):
                        ev, od = plsc.unpack(
                            plsc.bitcast(rows_v[base + kk, pl.ds(cc * 16, 16)],
                                         jnp.bfloat16),
                            format=plsc.PackFormat.INTERLEAVED)
                        acc[2 * cc] = acc[2 * cc] + wspl * ev
                        acc[2 * cc + 1] = acc[2 * cc + 1] + wspl * od
                obase = bb * C
                for cc in range(CG):
                    pos = obase + cc * 32 + 2 * lane
                    plsc.store_scatter(out_v, [pos], acc[2 * cc])
                    plsc.store_scatter(out_v, [pos + 1], acc[2 * cc + 1])
                return c2

            lax.fori_loop(0, CHUNK, bin_body, 0)
            pltpu.async_copy(out_v, out_hbm.at[pl.ds(bin0 * C, CHUNK * C)], sw)

        def pair_body(t, carry):
            g0 = t * 2
            pltpu.async_copy(table_hbm.at[idx_v.at[g0 + 1]], rows1, sg1)
            pltpu.make_async_copy(table_hbm.at[idx_v.at[0]], rows0, sg0).wait()

            @pl.when(t > 0)
            def _():
                pltpu.make_async_copy(out0, out_hbm.at[pl.ds(0, CHUNK * C)],
                                      sw0).wait()

            compute(g0, rows0, out0, sw0)

            @pl.when(t < NCHUNK // 2 - 1)
            def _():
                pltpu.async_copy(table_hbm.at[idx_v.at[g0 + 2]], rows0, sg0)

            pltpu.make_async_copy(table_hbm.at[idx_v.at[0]], rows1, sg1).wait()

            @pl.when(t > 0)
            def _():
                pltpu.make_async_copy(out1, out_hbm.at[pl.ds(0, CHUNK * C)],
                                      sw1).wait()

            compute(g0 + 1, rows1, out1, sw1)
            return carry

        lax.fori_loop(0, NCHUNK // 2, pair_body, 0)
        pltpu.make_async_copy(out0, out_hbm.at[pl.ds(0, CHUNK * C)], sw0).wait()
        pltpu.make_async_copy(out1, out_hbm.at[pl.ds(0, CHUNK * C)], sw1).wait()

    return k(table, idxs, ws)


def _xpose_body(in_ref, out_ref):
    x = in_ref[:].reshape(TR, BINS, C)
    out_ref[:] = jnp.transpose(x, (0, 2, 1))


def _xpose(rows):
    return pl.pallas_call(
        _xpose_body,
        grid=(R // TR,),
        in_specs=[pl.BlockSpec((TR * BINS, C), lambda i: (i, 0))],
        out_specs=pl.BlockSpec((TR, C, BINS), lambda i: (i, 0, 0)),
        out_shape=jax.ShapeDtypeStruct((R, C, BINS), jnp.float32),
    )(rows)


def kernel(_input, rois):
    f = jnp.transpose(_input, (0, 2, 3, 1)).reshape(NB * H * W, C)
    f = lax.bitcast_convert_type(
        f.astype(jnp.bfloat16).reshape(NB * H * W, C // 2, 2), jnp.int32)
    w2, idx2 = _coefs(rois, jnp.asarray(_MY), jnp.asarray(_MX))
    w_flat = jnp.pad(w2.reshape(J * K), (0, (JPAD - J) * K))
    idx_flat = jnp.pad(idx2.reshape(J * K), (0, (JPAD - J) * K))
    idx_2d = idx_flat.reshape(NW * NCHUNK, CHUNK * K)
    out_rows = _sc_gather(f, idx_2d, w_flat)
    out = _xpose(out_rows.reshape(JPAD, C))
    return out.reshape(R, C, OUT_HW, OUT_HW)


# speedup vs baseline: 2.8936x; 1.1951x over previous
import functools

import jax
import jax.numpy as jnp
import numpy as np
from jax import lax
from jax.experimental import pallas as pl
from jax.experimental.pallas import tpu as pltpu
from jax.experimental.pallas import tpu_sc as plsc

OUT_HW = 7
SR = 2
S = OUT_HW * SR
SCALE = 0.25
NB, C, H, W = 2, 256, 100, 100
R = 1000
BINS = OUT_HW * OUT_HW
K = 16
J = R * BINS
NW = 32
JPAD = 49152
BINS_PER_W = JPAD // NW
CHUNK = 8
NCHUNK = BINS_PER_W // CHUNK
CG = C // 32
TR = 8


def _selection_mats():
    my = np.zeros((2 * S, BINS * K), np.float32)
    mx = np.zeros((2 * S, BINS * K), np.float32)
    for p in range(OUT_HW):
        for q in range(OUT_HW):
            for i in range(SR):
                for jj in range(SR):
                    for cy in range(2):
                        for cx in range(2):
                            col = (p * OUT_HW + q) * K + (i * SR + jj) * 4 + cy * 2 + cx
                            my[cy * S + (SR * p + i), col] = 1.0
                            mx[cx * S + (SR * q + jj), col] = 1.0
    return my, mx


_MY, _MX = _selection_mats()


def _coef_body(rois_ref, my_ref, mx_ref, w_ref, idx_ref):
    r = rois_ref[:]
    b = r[:, 0:1]
    x1 = r[:, 1:2] * SCALE
    y1 = r[:, 2:3] * SCALE
    x2 = r[:, 3:4] * SCALE
    y2 = r[:, 4:5] * SCALE
    bin_w = jnp.maximum(x2 - x1, 1.0) / OUT_HW
    bin_h = jnp.maximum(y2 - y1, 1.0) / OUT_HW
    s = lax.broadcasted_iota(jnp.int32, (1, S), 1).astype(jnp.float32)
    p_ = jnp.floor(s * 0.5)
    off = p_ + ((s - 2.0 * p_) + 0.5) * 0.5
    gx = x1 + off * bin_w
    gy = y1 + off * bin_h

    def axis(coord, size):
        v = ((coord >= -1.0) & (coord <= float(size))).astype(jnp.float32)
        c = jnp.clip(coord, 0.0, float(size - 1))
        lo = jnp.minimum(jnp.floor(c), float(size - 2))
        f = c - lo
        return (jnp.concatenate([(1.0 - f) * v, f * v], axis=1),
                jnp.concatenate([lo, lo + 1.0], axis=1))

    wyc, iyc = axis(gy, H)
    wxc, ixc = axis(gx, W)
    my = my_ref[:]
    mx = mx_ref[:]
    wy_sel = jnp.dot(wyc, my, preferred_element_type=jnp.float32)
    wx_sel = jnp.dot(wxc, mx, preferred_element_type=jnp.float32)
    iy_sel = jnp.dot(iyc, my, preferred_element_type=jnp.float32)
    ix_sel = jnp.dot(ixc, mx, preferred_element_type=jnp.float32)
    w_ref[:] = 0.25 * wy_sel * wx_sel
    idx_ref[:] = (b * float(H * W) + iy_sel * float(W) + ix_sel).astype(jnp.int32)


def _coefs(rois, my, mx):
    return pl.pallas_call(
        _coef_body,
        out_shape=[jax.ShapeDtypeStruct((R, BINS * K), jnp.float32),
                   jax.ShapeDtypeStruct((R, BINS * K), jnp.int32)],
    )(rois, my, mx)


def _sc_gather(table, idxs, ws):
    mesh = plsc.VectorSubcoreMesh(core_axis_name="c", subcore_axis_name="s")

    @functools.partial(
        pl.kernel,
        out_type=jax.ShapeDtypeStruct((JPAD * C,), jnp.float32),
        mesh=mesh,
        compiler_params=pltpu.CompilerParams(needs_layout_passes=False),
        scratch_types=[
            pltpu.VMEM((NCHUNK, CHUNK * K), jnp.int32),
            pltpu.VMEM((BINS_PER_W * K,), jnp.float32),
            pltpu.VMEM((CHUNK * K, C // 2), jnp.int32),
            pltpu.VMEM((CHUNK * K, C // 2), jnp.int32),
            pltpu.VMEM((CHUNK * C,), jnp.float32),
            pltpu.VMEM((CHUNK * C,), jnp.float32),
            pltpu.SemaphoreType.DMA,
            pltpu.SemaphoreType.DMA,
            pltpu.SemaphoreType.DMA,
            pltpu.SemaphoreType.DMA,
        ],
    )
    def k(table_hbm, idx_hbm, w_hbm, out_hbm, idx_v, w_v,
          rows0, rows1, out0, out1, sg0, sg1, sw0, sw1):
        wid = lax.axis_index("s") * 2 + lax.axis_index("c")
        tile0 = wid * BINS_PER_W
        pltpu.sync_copy(idx_hbm.at[pl.ds(wid * NCHUNK, NCHUNK)], idx_v)
        pltpu.sync_copy(w_hbm.at[pl.ds(tile0 * K, BINS_PER_W * K)], w_v)
        pltpu.async_copy(table_hbm.at[idx_v.at[0]], rows0, sg0)

        lane = lax.broadcasted_iota(jnp.int32, (16,), 0)

        def compute(g, rows_v, out_v, sw):
            bin0 = tile0 + g * CHUNK
            wbase = g * (CHUNK * K)

            def bin_body(bb, c2):
                base = bb * K
                acc = [jnp.zeros((16,), jnp.float32) for _ in range(2 * CG)]
                for kk in range(K):
                    wspl = plsc.load_gather(
                        w_v, [jnp.full((16,), wbase + base + kk, jnp.int32)])
                    for cc in range(CG):
                        ev, od = plsc.unpack(
                            plsc.bitcast(rows_v[base + kk, pl.ds(cc * 16, 16)],
                                         jnp.bfloat16),
                            format=plsc.PackFormat.INTERLEAVED)
                        acc[2 * cc] = acc[2 * cc] + wspl * ev
                        acc[2 * cc + 1] = acc[2 * cc + 1] + wspl * od
                obase = bb * C
                for cc in range(CG):
                    out_v[pl.ds(obase + cc * 16, 16)] = acc[2 * cc]
                    out_v[pl.ds(obase + C // 2 + cc * 16, 16)] = acc[2 * cc + 1]
                return c2

            lax.fori_loop(0, CHUNK, bin_body, 0)
            pltpu.async_copy(out_v, out_hbm.at[pl.ds(bin0 * C, CHUNK * C)], sw)

        def pair_body(t, carry):
            g0 = t * 2
            pltpu.async_copy(table_hbm.at[idx_v.at[g0 + 1]], rows1, sg1)
            pltpu.make_async_copy(table_hbm.at[idx_v.at[0]], rows0, sg0).wait()

            @pl.when(t > 0)
            def _():
                pltpu.make_async_copy(out0, out_hbm.at[pl.ds(0, CHUNK * C)],
                                      sw0).wait()

            compute(g0, rows0, out0, sw0)

            @pl.when(t < NCHUNK // 2 - 1)
            def _():
                pltpu.async_copy(table_hbm.at[idx_v.at[g0 + 2]], rows0, sg0)

            pltpu.make_async_copy(table_hbm.at[idx_v.at[0]], rows1, sg1).wait()

            @pl.when(t > 0)
            def _():
                pltpu.make_async_copy(out1, out_hbm.at[pl.ds(0, CHUNK * C)],
                                      sw1).wait()

            compute(g0 + 1, rows1, out1, sw1)
            return carry

        lax.fori_loop(0, NCHUNK // 2, pair_body, 0)
        pltpu.make_async_copy(out0, out_hbm.at[pl.ds(0, CHUNK * C)], sw0).wait()
        pltpu.make_async_copy(out1, out_hbm.at[pl.ds(0, CHUNK * C)], sw1).wait()

    return k(table, idxs, ws)


def _xpose_body(in_ref, out_ref):
    x = in_ref[:].reshape(TR, BINS, C)
    out_ref[:] = jnp.transpose(x, (0, 2, 1))


def _xpose(rows):
    return pl.pallas_call(
        _xpose_body,
        grid=(R // TR,),
        in_specs=[pl.BlockSpec((TR * BINS, C), lambda i: (i, 0))],
        out_specs=pl.BlockSpec((TR, C, BINS), lambda i: (i, 0, 0)),
        out_shape=jax.ShapeDtypeStruct((R, C, BINS), jnp.float32),
    )(rows)


def kernel(_input, rois):
    f = jnp.transpose(_input, (0, 2, 3, 1)).reshape(NB * H * W, C)
    f_bf = f.astype(jnp.bfloat16)
    lo = lax.bitcast_convert_type(f_bf[:, :C // 2], jnp.uint16).astype(jnp.uint32)
    hi = lax.bitcast_convert_type(f_bf[:, C // 2:], jnp.uint16).astype(jnp.uint32)
    f = lax.bitcast_convert_type(lo | (hi << 16), jnp.int32)
    w2, idx2 = _coefs(rois, jnp.asarray(_MY), jnp.asarray(_MX))
    w_flat = jnp.pad(w2.reshape(J * K), (0, (JPAD - J) * K))
    idx_flat = jnp.pad(idx2.reshape(J * K), (0, (JPAD - J) * K))
    idx_2d = idx_flat.reshape(NW * NCHUNK, CHUNK * K)
    out_rows = _sc_gather(f, idx_2d, w_flat)
    out = _xpose(out_rows.reshape(JPAD, C))
    return out.reshape(R, C, OUT_HW, OUT_HW)
